# Initial kernel scaffold; baseline (speedup 1.0000x reference)
import jax
import jax.numpy as jnp
from jax.experimental import pallas as pl


def kernel(pooled_height, pooled_width, spatial_scale, features, rois):
    raise NotImplementedError("write your pallas kernel here")


# speedup vs baseline: 1334.9735x; 1334.9735x over previous
import functools

import jax
import jax.numpy as jnp
from jax import lax
from jax.experimental import pallas as pl
from jax.experimental.pallas import tpu as pltpu
from jax.experimental.pallas import tpu_sc as plsc

_NROI = 32
_PH = 14
_PW = 14
_BINS = _PH * _PW
_PADB = 224
_NC = 2
_NS = 16
_NW = _NC * _NS
_TOT = _NROI * _PADB
_CHUNK = _TOT // _NW
_LANES = 16


def _grid_body(m_ref, ilt_ref, irt_ref, ilb_ref, irb_ref,
               wlt_ref, wrt_ref, wrb_ref, wlb_ref, *, wm1, hm1, tabh, tabc):
    m00 = m_ref[:, 0:1]
    m01 = m_ref[:, 1:2]
    m02 = m_ref[:, 2:3]
    m10 = m_ref[:, 3:4]
    m11 = m_ref[:, 4:5]
    m12 = m_ref[:, 5:6]

    lane = lax.broadcasted_iota(jnp.int32, (_NROI, _PADB), 1)
    yi = lax.div(lane, _PW)
    xi = lane - yi * _PW
    x = xi.astype(jnp.float32)
    y = yi.astype(jnp.float32)
    xp = x + 1.0
    yp = y + 1.0

    p0 = m00 * x + m01 * y + m02
    p1 = m10 * x + m11 * y + m12
    p2 = m00 * x + m01 * yp + m02
    p3 = m10 * x + m11 * yp + m12
    p4 = m00 * xp + m01 * y + m02
    p5 = m10 * xp + m11 * y + m12
    p6 = m00 * xp + m01 * yp + m02
    p7 = m10 * xp + m11 * yp + m12

    left = jnp.maximum(jnp.round(jnp.minimum(jnp.minimum(p0, p2), jnp.minimum(p4, p6))), 0.0)
    right = jnp.minimum(jnp.round(jnp.maximum(jnp.maximum(p0, p2), jnp.maximum(p4, p6))), wm1)
    top = jnp.maximum(jnp.round(jnp.minimum(jnp.minimum(p1, p3), jnp.minimum(p5, p7))), 0.0)
    bottom = jnp.minimum(jnp.round(jnp.maximum(jnp.maximum(p1, p3), jnp.maximum(p5, p7))), hm1)

    bin_cx = (left + right) / 2.0
    bin_cy = (top + bottom) / 2.0
    fl_cx = jnp.floor(bin_cx)
    fl_cy = jnp.floor(bin_cy)
    rx = bin_cx - fl_cx
    ry = bin_cy - fl_cy

    wlt_ref[...] = (1.0 - rx) * (1.0 - ry)
    wrt_ref[...] = rx * (1.0 - ry)
    wrb_ref[...] = rx * ry
    wlb_ref[...] = (1.0 - rx) * ry

    ai_l = jnp.clip(fl_cx.astype(jnp.int32), 0, tabh - 1)
    ai_r = jnp.clip(jnp.ceil(bin_cx).astype(jnp.int32), 0, tabh - 1)
    bi_t = jnp.clip(fl_cy.astype(jnp.int32), 0, tabc - 1)
    bi_b = jnp.clip(jnp.ceil(bin_cy).astype(jnp.int32), 0, tabc - 1)

    ilt_ref[...] = ai_l * tabc + bi_t
    irt_ref[...] = ai_r * tabc + bi_t
    ilb_ref[...] = ai_l * tabc + bi_b
    irb_ref[...] = ai_r * tabc + bi_b


def _grid_call(m, wm1, hm1, tabh, tabc):
    shp_i = jax.ShapeDtypeStruct((_NROI, _PADB), jnp.int32)
    shp_f = jax.ShapeDtypeStruct((_NROI, _PADB), jnp.float32)
    return pl.pallas_call(
        functools.partial(_grid_body, wm1=wm1, hm1=hm1, tabh=tabh, tabc=tabc),
        out_shape=(shp_i, shp_i, shp_i, shp_i, shp_f, shp_f, shp_f, shp_f),
    )(m)


@functools.cache
def _make_sc_gather(tab_size):
    mesh = plsc.VectorSubcoreMesh(
        core_axis_name="c", subcore_axis_name="s",
        num_cores=_NC, num_subcores=_NS)

    @functools.partial(
        pl.kernel,
        out_type=jax.ShapeDtypeStruct((_TOT,), jnp.float32),
        mesh=mesh,
        compiler_params=pltpu.CompilerParams(needs_layout_passes=False),
        scratch_types=[
            pltpu.VMEM((tab_size,), jnp.float32),
            pltpu.VMEM((_CHUNK,), jnp.int32),
            pltpu.VMEM((_CHUNK,), jnp.int32),
            pltpu.VMEM((_CHUNK,), jnp.int32),
            pltpu.VMEM((_CHUNK,), jnp.int32),
            pltpu.VMEM((_CHUNK,), jnp.float32),
            pltpu.VMEM((_CHUNK,), jnp.float32),
            pltpu.VMEM((_CHUNK,), jnp.float32),
            pltpu.VMEM((_CHUNK,), jnp.float32),
            pltpu.VMEM((_CHUNK,), jnp.float32),
        ],
    )
    def sc_gather(tab_hbm, ilt_hbm, irt_hbm, ilb_hbm, irb_hbm,
                  wlt_hbm, wrt_hbm, wrb_hbm, wlb_hbm, out_hbm,
                  tab_v, ilt_v, irt_v, ilb_v, irb_v,
                  wlt_v, wrt_v, wrb_v, wlb_v, out_v):
        wid = lax.axis_index("s") * _NC + lax.axis_index("c")
        base = wid * _CHUNK
        pltpu.sync_copy(tab_hbm, tab_v)
        pltpu.sync_copy(ilt_hbm.at[pl.ds(base, _CHUNK)], ilt_v)
        pltpu.sync_copy(irt_hbm.at[pl.ds(base, _CHUNK)], irt_v)
        pltpu.sync_copy(ilb_hbm.at[pl.ds(base, _CHUNK)], ilb_v)
        pltpu.sync_copy(irb_hbm.at[pl.ds(base, _CHUNK)], irb_v)
        pltpu.sync_copy(wlt_hbm.at[pl.ds(base, _CHUNK)], wlt_v)
        pltpu.sync_copy(wrt_hbm.at[pl.ds(base, _CHUNK)], wrt_v)
        pltpu.sync_copy(wrb_hbm.at[pl.ds(base, _CHUNK)], wrb_v)
        pltpu.sync_copy(wlb_hbm.at[pl.ds(base, _CHUNK)], wlb_v)
        for j in range(_CHUNK // _LANES):
            sl = pl.ds(j * _LANES, _LANES)
            vlt = plsc.load_gather(tab_v, [ilt_v[sl]]).astype(jnp.int32).astype(jnp.float32)
            vrt = plsc.load_gather(tab_v, [irt_v[sl]]).astype(jnp.int32).astype(jnp.float32)
            vrb = plsc.load_gather(tab_v, [irb_v[sl]]).astype(jnp.int32).astype(jnp.float32)
            vlb = plsc.load_gather(tab_v, [ilb_v[sl]]).astype(jnp.int32).astype(jnp.float32)
            acc = vlt * wlt_v[sl]
            acc = acc + vrt * wrt_v[sl]
            acc = acc + vrb * wrb_v[sl]
            acc = acc + vlb * wlb_v[sl]
            out_v[sl] = acc
        pltpu.sync_copy(out_v, out_hbm.at[pl.ds(base, _CHUNK)])

    return sc_gather


def _bcast_body(p_ref, o_ref):
    o_ref[...] = jnp.broadcast_to(p_ref[...], o_ref.shape)


def _bcast_call(pooled_col, channel):
    n = pooled_col.shape[0]
    blocks = 8
    rows = n // blocks
    return pl.pallas_call(
        _bcast_body,
        grid=(blocks,),
        in_specs=[pl.BlockSpec((rows, 1), lambda i: (i, 0))],
        out_specs=pl.BlockSpec((rows, channel), lambda i: (i, 0)),
        out_shape=jax.ShapeDtypeStruct((n, channel), jnp.float32),
    )(pooled_col)


def kernel(pooled_height, pooled_width, spatial_scale, features, rois):
    width = features.shape[1]
    height = features.shape[2]
    channel = features.shape[3]
    f0 = features[0, 0]
    tabh, tabc = f0.shape

    phf = jnp.asarray(pooled_height).astype(jnp.float32)
    pwf = jnp.asarray(pooled_width).astype(jnp.float32)
    ssf = jnp.asarray(spatial_scale).astype(jnp.float32)

    roi_idx = jnp.concatenate(
        [jnp.array([0], dtype=jnp.int32), jnp.arange(_NROI - 1, dtype=jnp.int32)])
    r = rois[0, roi_idx, :].astype(jnp.float32)
    a1, a2, a3, a4, a5 = r[:, 1], r[:, 2], r[:, 3], r[:, 4], r[:, 5]
    m5 = a5 * 180.0 * 3.1415926535
    roi_pw = (a4 / a3) * pwf
    dx = -roi_pw / 2.0
    dy = -phf / 2.0
    sx = (a4 / roi_pw) * ssf
    sy = a3 / (phf * ssf)
    alpha = jnp.cos(m5)
    beta = jnp.sin(m5)
    m00 = alpha * sx
    m01 = beta * sy
    m02 = m00 * dx + m01 * dy + a1 * ssf
    m10 = -beta * sx
    m11 = alpha * sy
    m12 = m10 * dx + m11 * dy + a2 * ssf
    m = jnp.stack([m00, m01, m02, m10, m11, m12], axis=1)

    ilt, irt, ilb, irb, wlt, wrt, wrb, wlb = _grid_call(
        m, float(width - 1), float(height - 1), tabh, tabc)

    sc_gather = _make_sc_gather(tabh * tabc)
    pooled = sc_gather(
        f0.reshape(-1),
        ilt.reshape(-1), irt.reshape(-1), ilb.reshape(-1), irb.reshape(-1),
        wlt.reshape(-1), wrt.reshape(-1), wrb.reshape(-1), wlb.reshape(-1))

    pooled_col = pooled.reshape(_NROI, _PADB)[:, :_BINS].reshape(_NROI * _BINS, 1)
    out2d = _bcast_call(pooled_col, channel)
    return out2d.reshape(_NROI, _PH, _PW, channel)
